# Initial kernel scaffold; baseline (speedup 1.0000x reference)
import jax
import jax.numpy as jnp
from jax.experimental import pallas as pl


def kernel(xyz, rpe_table):
    raise NotImplementedError("write your pallas kernel here")


# speedup vs baseline: 3.8001x; 3.8001x over previous
import functools

import jax
import jax.numpy as jnp
from jax import lax
from jax.experimental import pallas as pl
from jax.experimental.pallas import tpu as pltpu
from jax.experimental.pallas import tpu_sc as plsc

PATCH = 48
HEADS = 16
POS_BND = 38
RPE_NUM = 2 * POS_BND + 1
TAB_WORDS = 3 * RPE_NUM * HEADS
POS_PER_B = PATCH * PATCH
XYZ_W_PER_B = POS_PER_B * 3
OUT_W_PER_B = HEADS * POS_PER_B
NC, NS = 2, 16
NW = NC * NS


def _sc_body(xyz_hbm, tab_hbm, out_hbm, tab_v, xyz_v, out_v, bpw):
    wid = lax.axis_index("s") * NC + lax.axis_index("c")
    pltpu.sync_copy(tab_hbm, tab_v)
    lane3 = lax.iota(jnp.int32, 16) * 3

    def batch_body(b, carry):
        gb = wid * bpw + b
        pltpu.sync_copy(xyz_hbm.at[pl.ds(gb * XYZ_W_PER_B, XYZ_W_PER_B)], xyz_v)

        def j_body(j, c):
            base = lane3 + j * (3 * 16)

            def prep(d):
                g = plsc.load_gather(xyz_v, [base + d])
                g = jnp.minimum(jnp.maximum(g, -POS_BND), POS_BND)
                return (g + (POS_BND + d * RPE_NUM)) * HEADS

            b0, b1, b2 = prep(0), prep(1), prep(2)
            for h in range(HEADS):
                v = (plsc.load_gather(tab_v, [b0 + h])
                     + plsc.load_gather(tab_v, [b1 + h])
                     + plsc.load_gather(tab_v, [b2 + h]))
                out_v[pl.ds(h * POS_PER_B + j * 16, 16)] = v
            return c

        lax.fori_loop(0, POS_PER_B // 16, j_body, 0)
        pltpu.sync_copy(out_v, out_hbm.at[pl.ds(gb * OUT_W_PER_B, OUT_W_PER_B)])
        return carry

    lax.fori_loop(0, bpw, batch_body, 0)


def kernel(xyz, rpe_table):
    B = xyz.shape[0]
    bpw = B // NW
    xyz_flat = xyz.astype(jnp.int32).reshape(-1)
    tab_flat = rpe_table.reshape(-1)
    mesh = plsc.VectorSubcoreMesh(core_axis_name="c", subcore_axis_name="s")
    run = pl.kernel(
        functools.partial(_sc_body, bpw=bpw),
        mesh=mesh,
        compiler_params=pltpu.CompilerParams(needs_layout_passes=False),
        out_type=jax.ShapeDtypeStruct((B * OUT_W_PER_B,), jnp.float32),
        scratch_types=[
            pltpu.VMEM((TAB_WORDS,), jnp.float32),
            pltpu.VMEM((XYZ_W_PER_B,), jnp.int32),
            pltpu.VMEM((OUT_W_PER_B,), jnp.float32),
        ],
    )
    out = run(xyz_flat, tab_flat)
    return out.reshape(B, HEADS, PATCH, PATCH)


# speedup vs baseline: 308.5183x; 81.1860x over previous
import functools

import jax
import jax.numpy as jnp
from jax import lax
from jax.experimental import pallas as pl
from jax.experimental.pallas import tpu as pltpu
from jax.experimental.pallas import tpu_sc as plsc

PATCH = 48
HEADS = 16
POS_BND = 38
RPE_NUM = 2 * POS_BND + 1
TAB_WORDS = 3 * RPE_NUM * HEADS
JT, JR = PATCH // 8, 8
BTILES, BL = 8, 128
NC, NS = 2, 16
NW = NC * NS
UNITS = PATCH * BTILES
UPW = UNITS // NW


JTH = JT // 2
NHALF = 2 * UPW


def _sc_body(xyz_hbm, tab_hbm, out_hbm, tab_v, xyz_v, out_v,
             isem0, isem1, osem0, osem1):
    wid = lax.axis_index("s") * NC + lax.axis_index("c")
    pltpu.sync_copy(tab_hbm, tab_v)
    isems = (isem0, isem1)
    osems = (osem0, osem1)

    def slices(k):
        uid = wid * UPW + k // 2
        i = uid // BTILES
        bt = uid % BTILES
        jslab = pl.ds((k % 2) * JTH, JTH)
        return (xyz_hbm.at[i, :, jslab, bt], out_hbm.at[:, i, jslab, bt])

    pltpu.make_async_copy(slices(0)[0], xyz_v.at[0], isem0).start()

    def kk_body(kk, carry):
        for b in range(2):
            k = 2 * kk + b

            @pl.when(k + 1 < NHALF)
            def _():
                pltpu.make_async_copy(
                    slices(k + 1)[0], xyz_v.at[1 - b], isems[1 - b]).start()

            pltpu.make_async_copy(slices(k)[0], xyz_v.at[b], isems[b]).wait()

            @pl.when(k >= 2)
            def _():
                pltpu.make_async_copy(
                    out_v.at[b], slices(k - 2)[1], osems[b]).wait()

            @plsc.parallel_loop(0, JTH * JR * (BL // 16), unroll=2)
            def vec_body(t):
                jt = t // (JR * (BL // 16))
                jr = (t // (BL // 16)) % JR
                cc = t % (BL // 16)
                sl = pl.ds(cc * 16, 16)

                def prep(d):
                    g = xyz_v[b, d, jt, jr, sl]
                    g = jnp.minimum(jnp.maximum(g, -POS_BND), POS_BND)
                    return g + (POS_BND + d * RPE_NUM)

                r0, r1, r2 = prep(0), prep(1), prep(2)
                for p in range(HEADS // 2):
                    pb = p * (3 * RPE_NUM)
                    s = (plsc.bitcast(plsc.load_gather(tab_v, [r0 + pb]),
                                      jnp.bfloat16)
                         + plsc.bitcast(plsc.load_gather(tab_v, [r1 + pb]),
                                        jnp.bfloat16)
                         + plsc.bitcast(plsc.load_gather(tab_v, [r2 + pb]),
                                        jnp.bfloat16))
                    lo, hi = plsc.unpack(s, format=plsc.PackFormat.INTERLEAVED)
                    out_v[b, 2 * p, jt, jr, sl] = lo
                    out_v[b, 2 * p + 1, jt, jr, sl] = hi

            pltpu.make_async_copy(out_v.at[b], slices(k)[1], osems[b]).start()
        return carry

    lax.fori_loop(0, UPW, kk_body, 0)
    pltpu.make_async_copy(out_v.at[0], slices(NHALF - 2)[1], osems[0]).wait()
    pltpu.make_async_copy(out_v.at[1], slices(NHALF - 1)[1], osems[1]).wait()


def kernel(xyz, rpe_table):
    B = xyz.shape[0]
    xyz6 = (xyz.astype(jnp.int32)
            .transpose(1, 3, 2, 0)
            .reshape(PATCH, 3, JT, JR, BTILES, BL)
            .transpose(0, 1, 2, 4, 3, 5))
    u = jax.lax.bitcast_convert_type(
        rpe_table.astype(jnp.bfloat16), jnp.uint16).astype(jnp.uint32)
    words = (u[:, 0::2] | (u[:, 1::2] << 16)).astype(jnp.int32)
    tab_flat = words.T.reshape(-1)
    mesh = plsc.VectorSubcoreMesh(core_axis_name="c", subcore_axis_name="s")
    run = pl.kernel(
        _sc_body,
        mesh=mesh,
        compiler_params=pltpu.CompilerParams(needs_layout_passes=False),
        out_type=jax.ShapeDtypeStruct((HEADS, PATCH, JT, BTILES, JR, BL),
                                      jnp.float32),
        scratch_types=[
            pltpu.VMEM((TAB_WORDS // 2,), jnp.int32),
            pltpu.VMEM((2, 3, JTH, JR, BL), jnp.int32),
            pltpu.VMEM((2, HEADS, JTH, JR, BL), jnp.float32),
            pltpu.SemaphoreType.DMA,
            pltpu.SemaphoreType.DMA,
            pltpu.SemaphoreType.DMA,
            pltpu.SemaphoreType.DMA,
        ],
    )
    out6 = run(xyz6, tab_flat)
    return (out6.transpose(0, 1, 2, 4, 3, 5)
            .reshape(HEADS, PATCH, PATCH, B)
            .transpose(3, 0, 1, 2))


# speedup vs baseline: 308.9331x; 1.0013x over previous
import jax
import jax.numpy as jnp
from jax import lax
from jax.experimental import pallas as pl
from jax.experimental.pallas import tpu as pltpu
from jax.experimental.pallas import tpu_sc as plsc

PATCH = 48
HEADS = 16
POS_BND = 38
RPE_NUM = 2 * POS_BND + 1
TAB_WORDS = 3 * RPE_NUM * HEADS
JT, JR = PATCH // 8, 8
BTILES, BL = 8, 128
NC, NS = 2, 16
NW = NC * NS
UNITS = PATCH * BTILES
UPW = UNITS // NW


JTH = JT // 2
NHALF = 2 * UPW


def _sc_body(xyz_hbm, tab_hbm, out_hbm, tab_v, xyz_v, out_v,
             isem0, isem1, osem0, osem1):
    wid = lax.axis_index("s") * NC + lax.axis_index("c")
    pltpu.sync_copy(tab_hbm, tab_v)
    isems = (isem0, isem1)
    osems = (osem0, osem1)

    def slices(k):
        uid = wid * UPW + k // 2
        i = uid // BTILES
        bt = uid % BTILES
        jslab = pl.ds((k % 2) * JTH, JTH)
        return (xyz_hbm.at[i, :, jslab, bt], out_hbm.at[:, i, jslab, bt])

    pltpu.make_async_copy(slices(0)[0], xyz_v.at[0], isem0).start()

    def kk_body(kk, carry):
        for b in range(2):
            k = 2 * kk + b

            @pl.when(k + 1 < NHALF)
            def _():
                pltpu.make_async_copy(
                    slices(k + 1)[0], xyz_v.at[1 - b], isems[1 - b]).start()

            pltpu.make_async_copy(slices(k)[0], xyz_v.at[b], isems[b]).wait()

            @pl.when(k >= 2)
            def _():
                pltpu.make_async_copy(
                    out_v.at[b], slices(k - 2)[1], osems[b]).wait()

            @plsc.parallel_loop(0, JTH * JR * (BL // 16), unroll=2)
            def vec_body(t):
                jt = t // (JR * (BL // 16))
                jr = (t // (BL // 16)) % JR
                cc = t % (BL // 16)
                sl = pl.ds(cc * 16, 16)

                def prep(d):
                    g = xyz_v[b, d, jt, jr, sl]
                    g = jnp.minimum(jnp.maximum(g, -POS_BND), POS_BND)
                    return g + (POS_BND + d * RPE_NUM)

                r0, r1, r2 = prep(0), prep(1), prep(2)
                for p in range(HEADS // 2):
                    pb = p * (3 * RPE_NUM)
                    s = (plsc.bitcast(plsc.load_gather(tab_v, [r0 + pb]),
                                      jnp.bfloat16)
                         + plsc.bitcast(plsc.load_gather(tab_v, [r1 + pb]),
                                        jnp.bfloat16)
                         + plsc.bitcast(plsc.load_gather(tab_v, [r2 + pb]),
                                        jnp.bfloat16))
                    lo, hi = plsc.unpack(s, format=plsc.PackFormat.INTERLEAVED)
                    out_v[b, 2 * p, jt, jr, sl] = lo
                    out_v[b, 2 * p + 1, jt, jr, sl] = hi

            pltpu.make_async_copy(out_v.at[b], slices(k)[1], osems[b]).start()
        return carry

    lax.fori_loop(0, UPW, kk_body, 0)
    pltpu.make_async_copy(out_v.at[0], slices(NHALF - 2)[1], osems[0]).wait()
    pltpu.make_async_copy(out_v.at[1], slices(NHALF - 1)[1], osems[1]).wait()


def kernel(xyz, rpe_table):
    B = xyz.shape[0]
    xyz6 = (xyz.astype(jnp.int32)
            .transpose(1, 3, 2, 0)
            .reshape(PATCH, 3, JT, JR, BTILES, BL)
            .transpose(0, 1, 2, 4, 3, 5))
    u = jax.lax.bitcast_convert_type(
        rpe_table.astype(jnp.bfloat16), jnp.uint16).astype(jnp.uint32)
    words = (u[:, 0::2] | (u[:, 1::2] << 16)).astype(jnp.int32)
    tab_flat = words.T.reshape(-1)
    mesh = plsc.VectorSubcoreMesh(core_axis_name="c", subcore_axis_name="s")
    run = pl.kernel(
        _sc_body,
        mesh=mesh,
        compiler_params=pltpu.CompilerParams(needs_layout_passes=False),
        out_type=jax.ShapeDtypeStruct((HEADS, PATCH, JT, BTILES, JR, BL),
                                      jnp.float32),
        scratch_types=[
            pltpu.VMEM((TAB_WORDS // 2,), jnp.int32),
            pltpu.VMEM((2, 3, JTH, JR, BL), jnp.int32),
            pltpu.VMEM((2, HEADS, JTH, JR, BL), jnp.float32),
            pltpu.SemaphoreType.DMA,
            pltpu.SemaphoreType.DMA,
            pltpu.SemaphoreType.DMA,
            pltpu.SemaphoreType.DMA,
        ],
    )
    out6 = run(xyz6, tab_flat)
    return (out6.transpose(0, 1, 2, 4, 3, 5)
            .reshape(HEADS, PATCH, PATCH, B)
            .transpose(3, 0, 1, 2))
